# baseline (speedup 1.0000x reference)
import functools

import numpy as np
import jax
import jax.numpy as jnp
from jax import lax
from jax.experimental import pallas as pl
from jax.experimental.pallas import tpu as pltpu
from jax.experimental.pallas import tpu_sc as plsc

_N = 8388608
_NBINS = 15
_WORKERS = 32
_PER_W = _N // _WORKERS
_CH = 16384
_NCH = _PER_W // _CH
_STEPS = _CH // 16

_BOUNDS = np.linspace(0.0, 1.0, _NBINS + 1).astype(np.float32)


def _sc_body(y_hbm, t_hbm, b_hbm, out_hbm, ybuf, tbuf, bvm, hist):
    c = lax.axis_index("c")
    s = lax.axis_index("s")
    wid = s * 2 + c
    base = wid * _PER_W

    pltpu.sync_copy(b_hbm, bvm)

    zero = jnp.zeros((16,), jnp.float32)
    for r in range(16):
        for blk in range(3):
            hist[r, pl.ds(blk * 16, 16)] = zero

    lanes = lax.iota(jnp.int32, 16)
    ones = jnp.ones((16,), jnp.float32)

    def chunk_body(ci, carry):
        off = base + ci * _CH
        pltpu.sync_copy(y_hbm.at[pl.ds(off, _CH)], ybuf)
        pltpu.sync_copy(t_hbm.at[pl.ds(off, _CH)], tbuf)

        def step(i, carry2):
            y = ybuf[pl.ds(i * 16, 16)]
            t = tbuf[pl.ds(i * 16, 16)]
            h = jnp.minimum((y * 15.0).astype(jnp.int32), 14)
            bl = plsc.load_gather(bvm, [h])
            bh = plsc.load_gather(bvm, [h + 1])
            up = (y > bh).astype(jnp.int32)
            dn = (y < bl).astype(jnp.int32)
            idx = h + up - dn
            eq_lo = y == bl
            eq_hi = y == bh
            ex = jnp.where(eq_hi, h + 1, h - 1)
            exm = (eq_lo | eq_hi) & (ex >= 0) & (ex <= 14)
            exc = jnp.clip(ex, 0, 15)
            plsc.addupdate_scatter(hist, [idx, lanes], ones)
            plsc.addupdate_scatter(hist, [idx, lanes + 16], y)
            plsc.addupdate_scatter(hist, [idx, lanes + 32], t)
            plsc.addupdate_scatter(hist, [exc, lanes], ones, mask=exm)
            plsc.addupdate_scatter(hist, [exc, lanes + 16], y, mask=exm)
            plsc.addupdate_scatter(hist, [exc, lanes + 32], t, mask=exm)
            return carry2

        return lax.fori_loop(0, _STEPS, step, carry)

    lax.fori_loop(0, _NCH, chunk_body, jnp.int32(0))
    pltpu.sync_copy(hist, out_hbm.at[wid])


def _epi_body(p_ref, o_ref):
    sums = jnp.sum(p_ref[...], axis=0)
    cnt = jnp.sum(sums[:, 0:16], axis=1, keepdims=True)
    conf = jnp.sum(sums[:, 16:32], axis=1, keepdims=True)
    acc = jnp.sum(sums[:, 32:48], axis=1, keepdims=True)
    n = jnp.float32(_N)
    safe = jnp.where(cnt > 0.0, cnt, 1.0)
    contrib = jnp.abs(conf / safe - acc / safe) * (cnt / n)
    binid = lax.broadcasted_iota(jnp.int32, (16, 1), 0)
    valid = (cnt > 40.0) & (binid < _NBINS)
    r = jnp.sum(jnp.where(valid, contrib, 0.0))
    o_ref[...] = jnp.reshape(r, (1, 1))


@jax.jit
def _rbece(y_pred, y_true):
    bounds = jnp.asarray(_BOUNDS)
    mesh = plsc.VectorSubcoreMesh(core_axis_name="c", subcore_axis_name="s")
    partials = pl.kernel(
        _sc_body,
        mesh=mesh,
        out_type=jax.ShapeDtypeStruct((_WORKERS, 16, 48), jnp.float32),
        scratch_types=[
            pltpu.VMEM((_CH,), jnp.float32),
            pltpu.VMEM((_CH,), jnp.float32),
            pltpu.VMEM((16,), jnp.float32),
            pltpu.VMEM((16, 48), jnp.float32),
        ],
        compiler_params=pltpu.CompilerParams(needs_layout_passes=False),
    )(y_pred, y_true, bounds)
    out = pl.pallas_call(
        _epi_body,
        out_shape=jax.ShapeDtypeStruct((1, 1), jnp.float32),
    )(partials)
    return out[0, 0]


def kernel(y_pred, y_true):
    return _rbece(y_pred, y_true)


# speedup vs baseline: 3.3238x; 3.3238x over previous
import functools

import numpy as np
import jax
import jax.numpy as jnp
from jax import lax
from jax.experimental import pallas as pl
from jax.experimental.pallas import tpu as pltpu
from jax.experimental.pallas import tpu_sc as plsc

_N = 8388608
_NBINS = 15
_WORKERS = 32
_PER_W = _N // _WORKERS
_CH = 16384
_NCH = _PER_W // _CH
_UNROLL = 4
_STEPS = _CH // (16 * _UNROLL)
_HC = 80 * _UNROLL

_BOUNDS = np.linspace(0.0, 1.0, _NBINS + 1).astype(np.float32)
_UPPER = np.concatenate([_BOUNDS[1:], [np.float32(2.0)]]).astype(np.float32)
_SCALE = np.float32(15.0) * (np.float32(1.0) - np.float32(2.0**-20))


def _sc_body(y_hbm, t_hbm, b_hbm, out_hbm, ybufs, tbufs, bvm, hist, sems):
    c = lax.axis_index("c")
    s = lax.axis_index("s")
    wid = s * 2 + c
    base = wid * _PER_W

    pltpu.sync_copy(b_hbm, bvm)

    zero = jnp.zeros((16,), jnp.float32)
    for r in range(16):
        for blk in range(_HC // 16):
            hist[r, pl.ds(blk * 16, 16)] = zero

    lanes = lax.iota(jnp.int32, 16)
    ones = jnp.ones((16,), jnp.float32)

    def start(ci, b):
        off = base + ci * _CH
        pltpu.async_copy(y_hbm.at[pl.ds(off, _CH)], ybufs[b], sems[2 * b])
        pltpu.async_copy(t_hbm.at[pl.ds(off, _CH)], tbufs[b], sems[2 * b + 1])

    def wait(b):
        pltpu.make_async_copy(y_hbm.at[pl.ds(0, _CH)], ybufs[b], sems[2 * b]).wait()
        pltpu.make_async_copy(t_hbm.at[pl.ds(0, _CH)], tbufs[b], sems[2 * b + 1]).wait()

    def compute(ybuf, tbuf):
        @plsc.parallel_loop(0, _STEPS, 1, unroll=2)
        def body(i):
            for u in range(_UNROLL):
                off = i * (16 * _UNROLL) + u * 16
                y = ybuf[pl.ds(off, 16)]
                t = tbuf[pl.ds(off, 16)]
                h = jnp.minimum((y * _SCALE).astype(jnp.int32), 14)
                bh = plsc.load_gather(bvm, [h])
                idx = h + (y >= bh).astype(jnp.int32)
                eq = y == bh
                cb = u * 80
                plsc.addupdate_scatter(hist, [idx, lanes + cb], ones)
                plsc.addupdate_scatter(hist, [idx, lanes + (cb + 16)], y)
                plsc.addupdate_scatter(hist, [idx, lanes + (cb + 32)], t)
                plsc.addupdate_scatter(hist, [h, lanes + (cb + 48)], ones, mask=eq)
                plsc.addupdate_scatter(hist, [h, lanes + (cb + 64)], t, mask=eq)

    start(0, 0)
    start(1, 1)

    def outer(jj, carry):
        ci = jj * 2
        wait(0)
        compute(ybufs[0], tbufs[0])

        @pl.when(jj < _NCH // 2 - 1)
        def _():
            start(ci + 2, 0)

        wait(1)
        compute(ybufs[1], tbufs[1])

        @pl.when(jj < _NCH // 2 - 1)
        def _():
            start(ci + 3, 1)

        return carry

    lax.fori_loop(0, _NCH // 2, outer, jnp.int32(0))

    pltpu.sync_copy(hist, out_hbm.at[wid])


def _epi_body(p_ref, u_ref, o_ref):
    sums = jnp.sum(p_ref[...], axis=0)
    cnt = jnp.zeros((16, 1), jnp.float32)
    conf = jnp.zeros((16, 1), jnp.float32)
    acc = jnp.zeros((16, 1), jnp.float32)
    ecnt = jnp.zeros((16, 1), jnp.float32)
    eacc = jnp.zeros((16, 1), jnp.float32)
    for u in range(_UNROLL):
        cb = u * 80
        cnt = cnt + jnp.sum(sums[:, cb:cb + 16], axis=1, keepdims=True)
        conf = conf + jnp.sum(sums[:, cb + 16:cb + 32], axis=1, keepdims=True)
        acc = acc + jnp.sum(sums[:, cb + 32:cb + 48], axis=1, keepdims=True)
        ecnt = ecnt + jnp.sum(sums[:, cb + 48:cb + 64], axis=1, keepdims=True)
        eacc = eacc + jnp.sum(sums[:, cb + 64:cb + 80], axis=1, keepdims=True)
    upper = u_ref[...]
    cnt = cnt + ecnt
    conf = conf + ecnt * upper
    acc = acc + eacc
    n = jnp.float32(_N)
    safe = jnp.where(cnt > 0.0, cnt, 1.0)
    contrib = jnp.abs(conf / safe - acc / safe) * (cnt / n)
    binid = lax.broadcasted_iota(jnp.int32, (16, 1), 0)
    valid = (cnt > 40.0) & (binid < _NBINS)
    r = jnp.sum(jnp.where(valid, contrib, 0.0))
    o_ref[...] = jnp.reshape(r, (1, 1))


@jax.jit
def _rbece(y_pred, y_true):
    upper = jnp.asarray(_UPPER)
    mesh = plsc.VectorSubcoreMesh(core_axis_name="c", subcore_axis_name="s")
    partials = pl.kernel(
        _sc_body,
        mesh=mesh,
        out_type=jax.ShapeDtypeStruct((_WORKERS, 16, _HC), jnp.float32),
        scratch_types=[
            [pltpu.VMEM((_CH,), jnp.float32), pltpu.VMEM((_CH,), jnp.float32)],
            [pltpu.VMEM((_CH,), jnp.float32), pltpu.VMEM((_CH,), jnp.float32)],
            pltpu.VMEM((16,), jnp.float32),
            pltpu.VMEM((16, _HC), jnp.float32),
            [pltpu.SemaphoreType.DMA] * 4,
        ],
        compiler_params=pltpu.CompilerParams(needs_layout_passes=False),
    )(y_pred, y_true, upper)
    out = pl.pallas_call(
        _epi_body,
        out_shape=jax.ShapeDtypeStruct((1, 1), jnp.float32),
    )(partials, upper.reshape(16, 1))
    return out[0, 0]


def kernel(y_pred, y_true):
    return _rbece(y_pred, y_true)


# speedup vs baseline: 3.8906x; 1.1705x over previous
import functools

import numpy as np
import jax
import jax.numpy as jnp
from jax import lax
from jax.experimental import pallas as pl
from jax.experimental.pallas import tpu as pltpu
from jax.experimental.pallas import tpu_sc as plsc

_N = 8388608
_NBINS = 15
_WORKERS = 32
_PER_W = _N // _WORKERS
_CH = 16384
_NCH = _PER_W // _CH
_UNROLL = 4
_STEPS = _CH // (16 * _UNROLL)
_HSTRIDE = 512
_HWORDS = 16 * _HSTRIDE

_BOUNDS = np.linspace(0.0, 1.0, _NBINS + 1).astype(np.float32)
_UPPER = np.concatenate([_BOUNDS[1:], [np.float32(2.0)]]).astype(np.float32)
_SCALE = np.float32(15.0) * (np.float32(1.0) - np.float32(2.0**-20))


def _sc_body(y_hbm, t_hbm, b_hbm, out_hbm, ybufs, tbufs, bvm, hist, sems):
    c = lax.axis_index("c")
    s = lax.axis_index("s")
    wid = s * 2 + c
    base = wid * _PER_W

    pltpu.sync_copy(b_hbm, bvm)

    zero = jnp.zeros((16,), jnp.float32)
    for blk in range(_HWORDS // 16):
        hist[pl.ds(blk * 16, 16)] = zero

    lanes = lax.iota(jnp.int32, 16)
    ones = jnp.ones((16,), jnp.float32)

    def start(ci, b):
        off = base + ci * _CH
        pltpu.async_copy(y_hbm.at[pl.ds(off, _CH)], ybufs[b], sems[2 * b])
        pltpu.async_copy(t_hbm.at[pl.ds(off, _CH)], tbufs[b], sems[2 * b + 1])

    def wait(b):
        pltpu.make_async_copy(y_hbm.at[pl.ds(0, _CH)], ybufs[b], sems[2 * b]).wait()
        pltpu.make_async_copy(t_hbm.at[pl.ds(0, _CH)], tbufs[b], sems[2 * b + 1]).wait()

    def compute(ybuf, tbuf):
        @plsc.parallel_loop(0, _STEPS, 1, unroll=2)
        def body(i):
            for u in range(_UNROLL):
                off = i * (16 * _UNROLL) + u * 16
                y = ybuf[pl.ds(off, 16)]
                t = tbuf[pl.ds(off, 16)]
                h = (y * _SCALE).astype(jnp.int32)
                bh = plsc.load_gather(bvm, [h])
                eq = y == bh
                cb = u * 80
                rh = (h << 9) | lanes
                ridx = rh + jnp.where(y >= bh, 512, 0)
                plsc.addupdate_scatter(hist, [ridx + cb], ones)
                plsc.addupdate_scatter(hist, [ridx + (cb + 16)], y)
                plsc.addupdate_scatter(hist, [ridx + (cb + 32)], t)
                plsc.addupdate_scatter(hist, [rh + (cb + 48)], ones, mask=eq)
                plsc.addupdate_scatter(hist, [rh + (cb + 64)], t, mask=eq)

    start(0, 0)
    start(1, 1)

    def outer(jj, carry):
        ci = jj * 2
        wait(0)
        compute(ybufs[0], tbufs[0])

        @pl.when(jj < _NCH // 2 - 1)
        def _():
            start(ci + 2, 0)

        wait(1)
        compute(ybufs[1], tbufs[1])

        @pl.when(jj < _NCH // 2 - 1)
        def _():
            start(ci + 3, 1)

        return carry

    lax.fori_loop(0, _NCH // 2, outer, jnp.int32(0))

    pltpu.sync_copy(hist, out_hbm.at[wid])


def _epi_body(p_ref, u_ref, o_ref):
    sums = jnp.sum(p_ref[...], axis=0)
    cnt = jnp.zeros((16, 1), jnp.float32)
    conf = jnp.zeros((16, 1), jnp.float32)
    acc = jnp.zeros((16, 1), jnp.float32)
    ecnt = jnp.zeros((16, 1), jnp.float32)
    eacc = jnp.zeros((16, 1), jnp.float32)
    for u in range(_UNROLL):
        cb = u * 80
        cnt = cnt + jnp.sum(sums[:, cb:cb + 16], axis=1, keepdims=True)
        conf = conf + jnp.sum(sums[:, cb + 16:cb + 32], axis=1, keepdims=True)
        acc = acc + jnp.sum(sums[:, cb + 32:cb + 48], axis=1, keepdims=True)
        ecnt = ecnt + jnp.sum(sums[:, cb + 48:cb + 64], axis=1, keepdims=True)
        eacc = eacc + jnp.sum(sums[:, cb + 64:cb + 80], axis=1, keepdims=True)
    upper = u_ref[...]
    cnt = cnt + ecnt
    conf = conf + ecnt * upper
    acc = acc + eacc
    n = jnp.float32(_N)
    safe = jnp.where(cnt > 0.0, cnt, 1.0)
    contrib = jnp.abs(conf / safe - acc / safe) * (cnt / n)
    binid = lax.broadcasted_iota(jnp.int32, (16, 1), 0)
    valid = (cnt > 40.0) & (binid < _NBINS)
    r = jnp.sum(jnp.where(valid, contrib, 0.0))
    o_ref[...] = jnp.reshape(r, (1, 1))


@jax.jit
def _rbece(y_pred, y_true):
    upper = jnp.asarray(_UPPER)
    mesh = plsc.VectorSubcoreMesh(core_axis_name="c", subcore_axis_name="s")
    partials = pl.kernel(
        _sc_body,
        mesh=mesh,
        out_type=jax.ShapeDtypeStruct((_WORKERS, _HWORDS), jnp.float32),
        scratch_types=[
            [pltpu.VMEM((_CH,), jnp.float32), pltpu.VMEM((_CH,), jnp.float32)],
            [pltpu.VMEM((_CH,), jnp.float32), pltpu.VMEM((_CH,), jnp.float32)],
            pltpu.VMEM((16,), jnp.float32),
            pltpu.VMEM((_HWORDS,), jnp.float32),
            [pltpu.SemaphoreType.DMA] * 4,
        ],
        compiler_params=pltpu.CompilerParams(needs_layout_passes=False),
    )(y_pred, y_true, upper)
    out = pl.pallas_call(
        _epi_body,
        out_shape=jax.ShapeDtypeStruct((1, 1), jnp.float32),
    )(partials.reshape(_WORKERS, 16, _HSTRIDE), upper.reshape(16, 1))
    return out[0, 0]


def kernel(y_pred, y_true):
    return _rbece(y_pred, y_true)


# speedup vs baseline: 4.4695x; 1.1488x over previous
import functools

import numpy as np
import jax
import jax.numpy as jnp
from jax import lax
from jax.experimental import pallas as pl
from jax.experimental.pallas import tpu as pltpu
from jax.experimental.pallas import tpu_sc as plsc

_N = 8388608
_NBINS = 15
_WORKERS = 32
_PER_W = _N // _WORKERS
_CH = 16384
_NCH = _PER_W // _CH
_UNROLL = 4
_STEPS = _CH // (16 * _UNROLL)
_HSTRIDE = 512
_HWORDS = 16 * _HSTRIDE

_BOUNDS = np.linspace(0.0, 1.0, _NBINS + 1).astype(np.float32)
_UPPER = np.concatenate([_BOUNDS[1:], [np.float32(2.0)]]).astype(np.float32)
_SCALE = np.float32(15.0) * (np.float32(1.0) - np.float32(2.0**-20))
_PACK = np.float32(2.0**20)


def _sc_body(y_hbm, t_hbm, b_hbm, out_hbm, ybufs, tbufs, bvm, hist, sems):
    c = lax.axis_index("c")
    s = lax.axis_index("s")
    wid = s * 2 + c
    base = wid * _PER_W

    pltpu.sync_copy(b_hbm, bvm)

    zero = jnp.zeros((16,), jnp.float32)
    for blk in range(_HWORDS // 16):
        hist[pl.ds(blk * 16, 16)] = zero

    lanes = lax.iota(jnp.int32, 16)
    ones = jnp.ones((16,), jnp.float32)
    bv = bvm[...]

    def start(ci, b):
        off = base + ci * _CH
        pltpu.async_copy(y_hbm.at[pl.ds(off, _CH)], ybufs[b], sems[2 * b])
        pltpu.async_copy(t_hbm.at[pl.ds(off, _CH)], tbufs[b], sems[2 * b + 1])

    def wait(b):
        pltpu.make_async_copy(y_hbm.at[pl.ds(0, _CH)], ybufs[b], sems[2 * b]).wait()
        pltpu.make_async_copy(t_hbm.at[pl.ds(0, _CH)], tbufs[b], sems[2 * b + 1]).wait()

    def compute(ybuf, tbuf):
        @plsc.parallel_loop(0, _STEPS, 1, unroll=2)
        def body(i):
            for u in range(_UNROLL):
                off = i * (16 * _UNROLL) + u * 16
                y = ybuf[pl.ds(off, 16)]
                t = tbuf[pl.ds(off, 16)]
                h = (y * _SCALE).astype(jnp.int32)
                bh = jnp.take_along_axis(bv, h, axis=0, mode="promise_in_bounds")
                eq = y == bh
                cb = u * 64
                rh = (h << 9) | lanes
                ridx = rh + jnp.where(y >= bh, 512, 0)
                plsc.addupdate_scatter(hist, [ridx + cb], ones)
                plsc.addupdate_scatter(hist, [ridx + (cb + 16)], y)
                plsc.addupdate_scatter(hist, [ridx + (cb + 32)], t)
                plsc.addupdate_scatter(hist, [rh + (cb + 48)], t + _PACK, mask=eq)

    start(0, 0)
    start(1, 1)

    def outer(jj, carry):
        ci = jj * 2
        wait(0)
        compute(ybufs[0], tbufs[0])

        @pl.when(jj < _NCH // 2 - 1)
        def _():
            start(ci + 2, 0)

        wait(1)
        compute(ybufs[1], tbufs[1])

        @pl.when(jj < _NCH // 2 - 1)
        def _():
            start(ci + 3, 1)

        return carry

    lax.fori_loop(0, _NCH // 2, outer, jnp.int32(0))

    pltpu.sync_copy(hist, out_hbm.at[wid])


def _epi_body(p_ref, u_ref, o_ref):
    sums = jnp.sum(p_ref[...], axis=0)
    cnt = jnp.zeros((16, 1), jnp.float32)
    conf = jnp.zeros((16, 1), jnp.float32)
    acc = jnp.zeros((16, 1), jnp.float32)
    epack = jnp.zeros((16, 1), jnp.float32)
    for u in range(_UNROLL):
        cb = u * 64
        cnt = cnt + jnp.sum(sums[:, cb:cb + 16], axis=1, keepdims=True)
        conf = conf + jnp.sum(sums[:, cb + 16:cb + 32], axis=1, keepdims=True)
        acc = acc + jnp.sum(sums[:, cb + 32:cb + 48], axis=1, keepdims=True)
        epack = epack + jnp.sum(sums[:, cb + 48:cb + 64], axis=1, keepdims=True)
    upper = u_ref[...]
    ecnt = jnp.trunc(epack * (1.0 / _PACK))
    eacc = epack - ecnt * _PACK
    cnt = cnt + ecnt
    conf = conf + ecnt * upper
    acc = acc + eacc
    n = jnp.float32(_N)
    safe = jnp.where(cnt > 0.0, cnt, 1.0)
    contrib = jnp.abs(conf / safe - acc / safe) * (cnt / n)
    binid = lax.broadcasted_iota(jnp.int32, (16, 1), 0)
    valid = (cnt > 40.0) & (binid < _NBINS)
    r = jnp.sum(jnp.where(valid, contrib, 0.0))
    o_ref[...] = jnp.reshape(r, (1, 1))


@jax.jit
def _rbece(y_pred, y_true):
    upper = jnp.asarray(_UPPER)
    mesh = plsc.VectorSubcoreMesh(core_axis_name="c", subcore_axis_name="s")
    partials = pl.kernel(
        _sc_body,
        mesh=mesh,
        out_type=jax.ShapeDtypeStruct((_WORKERS, _HWORDS), jnp.float32),
        scratch_types=[
            [pltpu.VMEM((_CH,), jnp.float32), pltpu.VMEM((_CH,), jnp.float32)],
            [pltpu.VMEM((_CH,), jnp.float32), pltpu.VMEM((_CH,), jnp.float32)],
            pltpu.VMEM((16,), jnp.float32),
            pltpu.VMEM((_HWORDS,), jnp.float32),
            [pltpu.SemaphoreType.DMA] * 4,
        ],
        compiler_params=pltpu.CompilerParams(needs_layout_passes=False),
    )(y_pred, y_true, upper)
    out = pl.pallas_call(
        _epi_body,
        out_shape=jax.ShapeDtypeStruct((1, 1), jnp.float32),
    )(partials.reshape(_WORKERS, 16, _HSTRIDE), upper.reshape(16, 1))
    return out[0, 0]


def kernel(y_pred, y_true):
    return _rbece(y_pred, y_true)


# speedup vs baseline: 4.5554x; 1.0192x over previous
import functools

import numpy as np
import jax
import jax.numpy as jnp
from jax import lax
from jax.experimental import pallas as pl
from jax.experimental.pallas import tpu as pltpu
from jax.experimental.pallas import tpu_sc as plsc

_N = 8388608
_NBINS = 15
_WORKERS = 32
_PER_W = _N // _WORKERS
_CH = 16384
_NCH = _PER_W // _CH
_UNROLL = 4
_STEPS = _CH // (16 * _UNROLL)
_HSTRIDE = 512
_HWORDS = 16 * _HSTRIDE

_BOUNDS = np.linspace(0.0, 1.0, _NBINS + 1).astype(np.float32)
_UPPER = np.concatenate([_BOUNDS[1:], [np.float32(2.0)]]).astype(np.float32)
_SCALE = np.float32(15.0) * (np.float32(1.0) - np.float32(2.0**-20))
_PACK = np.float32(2.0**20)


def _sc_body(y_hbm, t_hbm, b_hbm, out_hbm, ybufs, tbufs, bvm, hist, hsmall, sems):
    c = lax.axis_index("c")
    s = lax.axis_index("s")
    wid = s * 2 + c
    base = wid * _PER_W

    pltpu.sync_copy(b_hbm, bvm)

    zero = jnp.zeros((16,), jnp.float32)
    for blk in range(_HWORDS // 16):
        hist[pl.ds(blk * 16, 16)] = zero

    lanes = lax.iota(jnp.int32, 16)
    ones = jnp.ones((16,), jnp.float32)
    bv = bvm[...]

    def start(ci, b):
        off = base + ci * _CH
        pltpu.async_copy(y_hbm.at[pl.ds(off, _CH)], ybufs[b], sems[2 * b])
        pltpu.async_copy(t_hbm.at[pl.ds(off, _CH)], tbufs[b], sems[2 * b + 1])

    def wait(b):
        pltpu.make_async_copy(y_hbm.at[pl.ds(0, _CH)], ybufs[b], sems[2 * b]).wait()
        pltpu.make_async_copy(t_hbm.at[pl.ds(0, _CH)], tbufs[b], sems[2 * b + 1]).wait()

    def compute(ybuf, tbuf):
        @plsc.parallel_loop(0, _STEPS, 1, unroll=2)
        def body(i):
            for u in range(_UNROLL):
                off = i * (16 * _UNROLL) + u * 16
                y = ybuf[pl.ds(off, 16)]
                t = tbuf[pl.ds(off, 16)]
                h = (y * _SCALE).astype(jnp.int32)
                bh = jnp.take_along_axis(bv, h, axis=0, mode="promise_in_bounds")
                eq = y == bh
                cb = u * 64
                rh = (h << 9) | lanes
                ridx = rh + jnp.where(y >= bh, 512, 0)
                plsc.addupdate_scatter(hist, [ridx + cb], ones)
                plsc.addupdate_scatter(hist, [ridx + (cb + 16)], y)
                plsc.addupdate_scatter(hist, [ridx + (cb + 32)], t)
                plsc.addupdate_scatter(hist, [rh + (cb + 48)], t + _PACK, mask=eq)

    start(0, 0)
    start(1, 1)

    def outer(jj, carry):
        ci = jj * 2
        wait(0)
        compute(ybufs[0], tbufs[0])

        @pl.when(jj < _NCH // 2 - 1)
        def _():
            start(ci + 2, 0)

        wait(1)
        compute(ybufs[1], tbufs[1])

        @pl.when(jj < _NCH // 2 - 1)
        def _():
            start(ci + 3, 1)

        return carry

    lax.fori_loop(0, _NCH // 2, outer, jnp.int32(0))

    for r in range(16):
        base = r * _HSTRIDE
        for f in range(4):
            v = hist[pl.ds(base + f * 16, 16)]
            for u in range(1, _UNROLL):
                v = v + hist[pl.ds(base + u * 64 + f * 16, 16)]
            hsmall[r, pl.ds(f * 16, 16)] = v

    pltpu.sync_copy(hsmall, out_hbm.at[wid])


def _epi_body(p_ref, u_ref, o_ref):
    sums = jnp.sum(p_ref[...], axis=0)
    cnt = jnp.sum(sums[:, 0:16], axis=1, keepdims=True)
    conf = jnp.sum(sums[:, 16:32], axis=1, keepdims=True)
    acc = jnp.sum(sums[:, 32:48], axis=1, keepdims=True)
    epack = jnp.sum(sums[:, 48:64], axis=1, keepdims=True)
    upper = u_ref[...]
    ecnt = jnp.trunc(epack * (1.0 / _PACK))
    eacc = epack - ecnt * _PACK
    cnt = cnt + ecnt
    conf = conf + ecnt * upper
    acc = acc + eacc
    n = jnp.float32(_N)
    safe = jnp.where(cnt > 0.0, cnt, 1.0)
    contrib = jnp.abs(conf / safe - acc / safe) * (cnt / n)
    binid = lax.broadcasted_iota(jnp.int32, (16, 1), 0)
    valid = (cnt > 40.0) & (binid < _NBINS)
    r = jnp.sum(jnp.where(valid, contrib, 0.0))
    o_ref[...] = jnp.reshape(r, (1, 1))


@jax.jit
def _rbece(y_pred, y_true):
    upper = jnp.asarray(_UPPER)
    mesh = plsc.VectorSubcoreMesh(core_axis_name="c", subcore_axis_name="s")
    partials = pl.kernel(
        _sc_body,
        mesh=mesh,
        out_type=jax.ShapeDtypeStruct((_WORKERS, 16, 64), jnp.float32),
        scratch_types=[
            [pltpu.VMEM((_CH,), jnp.float32), pltpu.VMEM((_CH,), jnp.float32)],
            [pltpu.VMEM((_CH,), jnp.float32), pltpu.VMEM((_CH,), jnp.float32)],
            pltpu.VMEM((16,), jnp.float32),
            pltpu.VMEM((_HWORDS,), jnp.float32),
            pltpu.VMEM((16, 64), jnp.float32),
            [pltpu.SemaphoreType.DMA] * 4,
        ],
        compiler_params=pltpu.CompilerParams(needs_layout_passes=False),
    )(y_pred, y_true, upper)
    out = pl.pallas_call(
        _epi_body,
        out_shape=jax.ShapeDtypeStruct((1, 1), jnp.float32),
    )(partials, upper.reshape(16, 1))
    return out[0, 0]


def kernel(y_pred, y_true):
    return _rbece(y_pred, y_true)
